# baseline (speedup 1.0000x reference)
import functools

import jax
import jax.numpy as jnp
from jax.experimental import pallas as pl
from jax.experimental.pallas import tpu as pltpu

_N = 10000
_E = 160000
_K = 5


def _elu(x):
    return jnp.where(x > 0, x, jnp.exp(jnp.minimum(x, 0.0)) - 1.0)


def _row_call(body, blocked, full, out_shapes, blk):
    n = blocked[0].shape[0]
    grid = (n // blk,)

    def _blk_spec(d):
        return pl.BlockSpec((blk, d), lambda i: (i, 0))

    def _full_spec(shape):
        return pl.BlockSpec(shape, lambda i: tuple(0 for _ in shape))

    in_specs = [_blk_spec(a.shape[1]) for a in blocked] + [
        _full_spec(a.shape) for a in full
    ]
    out_specs = [_blk_spec(s[1]) for s in out_shapes]
    out_shape = [jax.ShapeDtypeStruct(s, jnp.float32) for s in out_shapes]
    return pl.pallas_call(
        body,
        grid=grid,
        in_specs=in_specs,
        out_specs=out_specs,
        out_shape=out_shape,
    )(*blocked, *full)


def _m1_body(x_ref, w_ref, am_src_ref, am_dst_ref, h_ref, as_ref, ad_ref):
    h = jnp.dot(x_ref[...], w_ref[...], preferred_element_type=jnp.float32)
    h_ref[...] = h
    as_ref[...] = jnp.dot(h, am_src_ref[...], preferred_element_type=jnp.float32)
    ad_ref[...] = jnp.dot(h, am_dst_ref[...], preferred_element_type=jnp.float32)


def _m2_body(agg_ref, b_ref, sw_ref, root_ref, sb_ref, z_ref, r_ref):
    x1 = _elu(agg_ref[...] + b_ref[...])
    z_ref[...] = jnp.dot(x1, sw_ref[...], preferred_element_type=jnp.float32)
    r_ref[...] = (
        jnp.dot(x1, root_ref[...], preferred_element_type=jnp.float32) + sb_ref[...]
    )


def _m3_body(sagg_ref, r_ref, w_ref, am_src_ref, am_dst_ref, h_ref, as_ref, ad_ref):
    x2 = _elu(sagg_ref[...] + r_ref[...])
    h = jnp.dot(x2, w_ref[...], preferred_element_type=jnp.float32)
    h_ref[...] = h
    as_ref[...] = jnp.dot(h, am_src_ref[...], preferred_element_type=jnp.float32)
    ad_ref[...] = jnp.dot(h, am_dst_ref[...], preferred_element_type=jnp.float32)


def _m5_body(sagg_ref, r_ref, o_ref):
    o_ref[...] = jax.nn.sigmoid(sagg_ref[...] + r_ref[...])


def _att_matrix(att):
    heads, ch = att.shape
    eye = jnp.eye(heads, dtype=att.dtype)
    return (att[:, :, None] * eye[:, None, :]).reshape(heads * ch, heads)


def _spline_basis(pseudo):
    v = pseudo * (_K - 1)
    vf = jnp.floor(v)
    frac = v - vf
    vi = vf.astype(jnp.int32)
    bs, ids = [], []
    for s in range(4):
        k0, k1 = s % 2, s // 2
        b0 = frac[:, 0] if k0 else 1.0 - frac[:, 0]
        b1 = frac[:, 1] if k1 else 1.0 - frac[:, 1]
        i0 = jnp.clip(vi[:, 0] + k0, 0, _K - 1)
        i1 = jnp.clip(vi[:, 1] + k1, 0, _K - 1)
        bs.append(b0 * b1)
        ids.append(i0 + _K * i1)
    return jnp.stack(bs, 1), jnp.stack(ids, 1)


def _gat_edge_xla(h, a_src, a_dst, src, dst, heads, ch):
    n = h.shape[0]
    e = jax.nn.leaky_relu(a_src[src] + a_dst[dst], 0.2)
    emax = jax.ops.segment_max(e, dst, num_segments=n)
    emax = jnp.where(jnp.isfinite(emax), emax, 0.0)
    ex = jnp.exp(e - emax[dst])
    denom = jax.ops.segment_sum(ex, dst, num_segments=n)
    alpha = ex / (denom[dst] + 1e-16)
    h3 = h.reshape(n, heads, ch)
    msg = h3[src] * alpha[..., None]
    agg = jax.ops.segment_sum(msg, dst, num_segments=n)
    return agg.reshape(n, heads * ch)


def _spline_edge_xla(z, src, dst, basis, kidx, n, d):
    zr = z.reshape(n, _K * _K, d)
    g = zr[src[:, None], kidx]
    msg = (basis[..., None] * g).sum(1)
    agg = jax.ops.segment_max(msg, dst, num_segments=n)
    return jnp.where(jnp.isfinite(agg), agg, 0.0)


def kernel(x, edge_index, edge_attr, W1, att_src1, att_dst1, b1, sw1, root1,
           sb1, W2, att_src2, att_dst2, b2, sw2, root2, sb2):
    n = _N
    src_e, dst_e = edge_index[0], edge_index[1]
    loop = jnp.arange(n, dtype=edge_index.dtype)
    src = jnp.concatenate([src_e, loop])
    dst = jnp.concatenate([dst_e, loop])

    am_s1 = _att_matrix(att_src1)
    am_d1 = _att_matrix(att_dst1)
    h1, as1, ad1 = _row_call(
        _m1_body, [x], [W1, am_s1, am_d1],
        [(n, 256), (n, 8), (n, 8)], blk=2000)
    agg1 = _gat_edge_xla(h1, as1, ad1, src, dst, 8, 32)

    sw1f = sw1.reshape(_K * _K, 256, 64).transpose(1, 0, 2).reshape(256, _K * _K * 64)
    z1, r1 = _row_call(
        _m2_body, [agg1], [b1.reshape(1, 256), sw1f, root1, sb1.reshape(1, 64)],
        [(n, _K * _K * 64), (n, 64)], blk=1000)
    basis, kidx = _spline_basis(edge_attr)
    sagg1 = _spline_edge_xla(z1, src_e, dst_e, basis, kidx, n, 64)

    am_s2 = _att_matrix(att_src2)
    am_d2 = _att_matrix(att_dst2)
    h2, as2, ad2 = _row_call(
        _m3_body, [sagg1, r1], [W2, am_s2, am_d2],
        [(n, 512), (n, 8), (n, 8)], blk=2000)
    agg2 = _gat_edge_xla(h2, as2, ad2, src, dst, 8, 64)

    sw2f = sw2.reshape(_K * _K, 512, 1).transpose(1, 0, 2).reshape(512, _K * _K)
    z2, r2 = _row_call(
        _m2_body, [agg2], [b2.reshape(1, 512), sw2f, root2, sb2.reshape(1, 1)],
        [(n, _K * _K), (n, 1)], blk=1000)
    sagg2 = _spline_edge_xla(z2, src_e, dst_e, basis, kidx, n, 1)

    out = _row_call(_m5_body, [sagg2, r2], [], [(n, 1)], blk=2000)[0]
    return out
